# baseline (speedup 1.0000x reference)
import functools

import jax
import jax.numpy as jnp
from jax import lax
from jax.experimental import pallas as pl
from jax.experimental.pallas import tpu as pltpu
from jax.experimental.pallas import tpu_sc as plsc

N_E = 8192
E_DIM = 256
BETA = 0.25
N_TOK = 16384
B_BLK = 256

_WINDOWS = ((0, 2736), (2736, 5472), (5472, 8192))


def _bf16(x):
    return x.astype(jnp.bfloat16).astype(jnp.float32)


def _argmin_body(z_ref, zz_ref, et_ref, ee_ref, idx_ref):
    zb = z_ref[...].astype(jnp.bfloat16)
    mm = jnp.dot(zb, et_ref[...], preferred_element_type=jnp.float32)
    d = (zz_ref[...] + ee_ref[...]) - 2.0 * mm
    (l1, h1), (l2, h2), (l3, h3) = _WINDOWS
    m1 = jnp.min(d[:, l1:h1], axis=1, keepdims=True)
    m2 = jnp.min(d[:, l2:h2], axis=1, keepdims=True)
    m3 = jnp.min(d[:, l3:h3], axis=1, keepdims=True)
    b1 = _bf16(m1)
    t2 = m2 < b1
    b2 = _bf16(jnp.where(t2, m2, b1))
    t3 = m3 < b2
    niota = lax.broadcasted_iota(jnp.int32, d.shape, 1)
    mstar = jnp.where(niota < h1, m1, jnp.where(niota < h2, m2, m3))
    key = jnp.where(d == mstar, niota, N_E)
    i1 = jnp.min(key[:, l1:h1], axis=1, keepdims=True)
    i2 = jnp.min(key[:, l2:h2], axis=1, keepdims=True)
    i3 = jnp.min(key[:, l3:h3], axis=1, keepdims=True)
    idx = jnp.where(t3, i3, jnp.where(t2, i2, i1))
    idx_ref[0, 0, :] = idx[:, 0]


def _distance_argmin(z_flat, zz, e_t, ee):
    n_tok = z_flat.shape[0]
    n_blk = n_tok // B_BLK
    idx3 = pl.pallas_call(
        _argmin_body,
        grid=(n_blk,),
        in_specs=[
            pl.BlockSpec((B_BLK, E_DIM), lambda i: (i, 0)),
            pl.BlockSpec((B_BLK, 1), lambda i: (i, 0)),
            pl.BlockSpec((E_DIM, N_E), lambda i: (0, 0)),
            pl.BlockSpec((1, N_E), lambda i: (0, 0)),
        ],
        out_specs=pl.BlockSpec((1, 1, B_BLK), lambda i: (i, 0, 0)),
        out_shape=jax.ShapeDtypeStruct((n_blk, 1, B_BLK), jnp.int32),
    )(z_flat, zz, e_t, ee)
    return idx3.reshape(n_tok)


def _sc_gather(embedding_weight, idx):
    n_rows = idx.shape[0]
    info = plsc.get_sparse_core_info()
    nw = info.num_cores * info.num_subcores
    b_per_w = n_rows // nw
    chunk = 128
    n_chunks = b_per_w // chunk
    mesh = plsc.VectorSubcoreMesh(core_axis_name="c", subcore_axis_name="s")

    @functools.partial(
        pl.kernel,
        mesh=mesh,
        out_type=jax.ShapeDtypeStruct((n_rows, E_DIM), jnp.float32),
        scratch_types=[
            pltpu.VMEM((chunk,), jnp.int32),
            pltpu.VMEM((chunk, E_DIM), jnp.float32),
            pltpu.SemaphoreType.DMA,
        ],
    )
    def gather_kernel(table_hbm, idx_hbm, out_hbm, idx_v, rows_v, sem):
        wid = lax.axis_index("s") * info.num_cores + lax.axis_index("c")
        base = wid * b_per_w

        def body(c, carry):
            off = base + c * chunk
            pltpu.sync_copy(idx_hbm.at[pl.ds(off, chunk)], idx_v)
            pltpu.async_copy(table_hbm.at[idx_v], rows_v, sem).wait()
            pltpu.sync_copy(rows_v, out_hbm.at[pl.ds(off, chunk)])
            return carry

        lax.fori_loop(0, n_chunks, body, 0)

    return gather_kernel(embedding_weight, idx)


def _ste_body(z_ref, zqlo_ref, zqhi_ref, zqo_ref, loss_ref):
    h = pl.program_id(0)
    zq = jnp.where(h == 0, zqlo_ref[...], zqhi_ref[...])
    zqo_ref[...] = zq
    w = zq - z_ref[...]
    w2 = w * w
    loss_ref[...] = w2 + BETA * w2


def _ste(z_flat, zq_lo, zq_hi):
    rows = 2048
    per_half = (N_TOK // 2) // rows
    return pl.pallas_call(
        _ste_body,
        grid=(2, per_half),
        in_specs=[
            pl.BlockSpec((rows, E_DIM), lambda h, j: (h * per_half + j, 0)),
            pl.BlockSpec((rows, E_DIM), lambda h, j: (jnp.where(h == 0, j, per_half - 1), 0)),
            pl.BlockSpec((rows, E_DIM), lambda h, j: (jnp.where(h == 0, 0, j), 0)),
        ],
        out_specs=[
            pl.BlockSpec((rows, E_DIM), lambda h, j: (h * per_half + j, 0)),
            pl.BlockSpec((rows, E_DIM), lambda h, j: (h * per_half + j, 0)),
        ],
        out_shape=[
            jax.ShapeDtypeStruct((N_TOK, E_DIM), jnp.float32),
            jax.ShapeDtypeStruct((N_TOK, E_DIM), jnp.float32),
        ],
    )(z_flat, zq_lo, zq_hi)


def kernel(z, embedding_weight):
    z_flat = z.reshape(-1, E_DIM)
    zz = jnp.sum(z_flat**2, axis=1, keepdims=True)
    ee = jnp.sum(embedding_weight**2, axis=1)[None, :]
    e_t = embedding_weight.T.astype(jnp.bfloat16)
    h = N_TOK // 2
    idx_lo = _distance_argmin(z_flat[:h], zz[:h], e_t, ee)
    idx_hi = _distance_argmin(z_flat[h:], zz[h:], e_t, ee)
    zq_lo = _sc_gather(embedding_weight, idx_lo)
    zq_hi = _sc_gather(embedding_weight, idx_hi)
    zqo, loss = _ste(z_flat, zq_lo, zq_hi)
    idx = jnp.concatenate([idx_lo, idx_hi])
    return (zqo.reshape(z.shape), idx, loss.reshape(z.shape))


# speedup vs baseline: 1.0213x; 1.0213x over previous
import functools

import jax
import jax.numpy as jnp
from jax import lax
from jax.experimental import pallas as pl
from jax.experimental.pallas import tpu as pltpu
from jax.experimental.pallas import tpu_sc as plsc

N_E = 8192
E_DIM = 256
BETA = 0.25
N_TOK = 16384
B_BLK = 256


_WINDOWS = ((0, 2736), (2736, 5472), (5472, 8192))


def _bf16(x):
    return x.astype(jnp.bfloat16).astype(jnp.float32)


def _argmin_body(z_ref, zz_ref, et_ref, ee_ref, idx_ref):
    zb = z_ref[...].astype(jnp.bfloat16)
    mm = jnp.dot(zb, et_ref[...], preferred_element_type=jnp.float32)
    d = (zz_ref[...] + ee_ref[...]) - 2.0 * mm
    (l1, h1), (l2, h2), (l3, h3) = _WINDOWS
    m1 = jnp.min(d[:, l1:h1], axis=1, keepdims=True)
    m2 = jnp.min(d[:, l2:h2], axis=1, keepdims=True)
    m3 = jnp.min(d[:, l3:h3], axis=1, keepdims=True)
    b1 = _bf16(m1)
    t2 = m2 < b1
    b2 = _bf16(jnp.where(t2, m2, b1))
    t3 = m3 < b2
    niota = lax.broadcasted_iota(jnp.int32, d.shape, 1)

    def fidx(lo, hi, m):
        key = jnp.where(d[:, lo:hi] == m, niota[:, lo:hi], N_E)
        return jnp.min(key, axis=1, keepdims=True)

    i1 = fidx(l1, h1, m1)
    i2 = fidx(l2, h2, m2)
    i3 = fidx(l3, h3, m3)
    idx = jnp.where(t3, i3, jnp.where(t2, i2, i1))
    idx_ref[0, 0, :] = idx[:, 0]


def _distance_argmin(z_flat, zz, e_t, ee):
    n_blk = N_TOK // B_BLK
    idx3 = pl.pallas_call(
        _argmin_body,
        grid=(n_blk,),
        in_specs=[
            pl.BlockSpec((B_BLK, E_DIM), lambda i: (i, 0)),
            pl.BlockSpec((B_BLK, 1), lambda i: (i, 0)),
            pl.BlockSpec((E_DIM, N_E), lambda i: (0, 0)),
            pl.BlockSpec((1, N_E), lambda i: (0, 0)),
        ],
        out_specs=pl.BlockSpec((1, 1, B_BLK), lambda i: (i, 0, 0)),
        out_shape=jax.ShapeDtypeStruct((n_blk, 1, B_BLK), jnp.int32),
    )(z_flat, zz, e_t, ee)
    return idx3.reshape(N_TOK)


def _sc_gather(embedding_weight, idx):
    info = plsc.get_sparse_core_info()
    nw = info.num_cores * info.num_subcores
    b_per_w = N_TOK // nw
    chunk = 128
    n_chunks = b_per_w // chunk
    mesh = plsc.VectorSubcoreMesh(core_axis_name="c", subcore_axis_name="s")

    @functools.partial(
        pl.kernel,
        mesh=mesh,
        out_type=jax.ShapeDtypeStruct((N_TOK, E_DIM), jnp.float32),
        scratch_types=[
            pltpu.VMEM((chunk,), jnp.int32),
            pltpu.VMEM((chunk, E_DIM), jnp.float32),
            pltpu.SemaphoreType.DMA,
        ],
    )
    def gather_kernel(table_hbm, idx_hbm, out_hbm, idx_v, rows_v, sem):
        wid = lax.axis_index("s") * info.num_cores + lax.axis_index("c")
        base = wid * b_per_w

        def body(c, _):
            off = base + c * chunk
            pltpu.sync_copy(idx_hbm.at[pl.ds(off, chunk)], idx_v)
            pltpu.async_copy(table_hbm.at[idx_v], rows_v, sem).wait()
            pltpu.sync_copy(rows_v, out_hbm.at[pl.ds(off, chunk)])
            return _

        lax.fori_loop(0, n_chunks, body, 0)

    return gather_kernel(embedding_weight, idx)


def _loss_body(z_ref, zq_ref, loss_ref):
    w = zq_ref[...] - z_ref[...]
    w2 = w * w
    loss_ref[...] = w2 + BETA * w2


def _loss(z_flat, zq_flat):
    rows = 2048
    n_blk = N_TOK // rows
    return pl.pallas_call(
        _loss_body,
        grid=(n_blk,),
        in_specs=[
            pl.BlockSpec((rows, E_DIM), lambda i: (i, 0)),
            pl.BlockSpec((rows, E_DIM), lambda i: (i, 0)),
        ],
        out_specs=pl.BlockSpec((rows, E_DIM), lambda i: (i, 0)),
        out_shape=jax.ShapeDtypeStruct((N_TOK, E_DIM), jnp.float32),
    )(z_flat, zq_flat)


def kernel(z, embedding_weight):
    z_flat = z.reshape(-1, E_DIM)
    zz = jnp.sum(z_flat**2, axis=1, keepdims=True)
    ee = jnp.sum(embedding_weight**2, axis=1)[None, :]
    e_t = embedding_weight.T.astype(jnp.bfloat16)
    idx = _distance_argmin(z_flat, zz, e_t, ee)
    zq_flat = _sc_gather(embedding_weight, idx)
    loss = _loss(z_flat, zq_flat)
    return (zq_flat.reshape(z.shape), idx, loss.reshape(z.shape))


# speedup vs baseline: 1.0889x; 1.0662x over previous
import functools

import jax
import jax.numpy as jnp
from jax import lax
from jax.experimental import pallas as pl
from jax.experimental.pallas import tpu as pltpu
from jax.experimental.pallas import tpu_sc as plsc

N_E = 8192
E_DIM = 256
BETA = 0.25
N_TOK = 16384
B_BLK = 256


_WINDOWS = ((0, 2736), (2736, 5472), (5472, 8192))


def _bf16(x):
    return x.astype(jnp.bfloat16).astype(jnp.float32)


def _argmin_body(z_ref, zz_ref, et_ref, ee_ref, idx_ref):
    zb = z_ref[...].astype(jnp.bfloat16)
    mm = jnp.dot(zb, et_ref[...], preferred_element_type=jnp.float32)
    d = (zz_ref[...] + ee_ref[...]) - 2.0 * mm
    (l1, h1), (l2, h2), (l3, h3) = _WINDOWS
    m1 = jnp.min(d[:, l1:h1], axis=1, keepdims=True)
    m2 = jnp.min(d[:, l2:h2], axis=1, keepdims=True)
    m3 = jnp.min(d[:, l3:h3], axis=1, keepdims=True)
    b1 = _bf16(m1)
    t2 = m2 < b1
    b2 = _bf16(jnp.where(t2, m2, b1))
    t3 = m3 < b2
    niota = lax.broadcasted_iota(jnp.int32, d.shape, 1)
    mstar = jnp.where(niota < h1, m1, jnp.where(niota < h2, m2, m3))
    key = jnp.where(d == mstar, niota, N_E)
    i1 = jnp.min(key[:, l1:h1], axis=1, keepdims=True)
    i2 = jnp.min(key[:, l2:h2], axis=1, keepdims=True)
    i3 = jnp.min(key[:, l3:h3], axis=1, keepdims=True)
    idx = jnp.where(t3, i3, jnp.where(t2, i2, i1))
    idx_ref[0, 0, :] = idx[:, 0]


def _distance_argmin(z_flat, zz, e_t, ee):
    n_blk = N_TOK // B_BLK
    idx3 = pl.pallas_call(
        _argmin_body,
        grid=(n_blk,),
        in_specs=[
            pl.BlockSpec((B_BLK, E_DIM), lambda i: (i, 0)),
            pl.BlockSpec((B_BLK, 1), lambda i: (i, 0)),
            pl.BlockSpec((E_DIM, N_E), lambda i: (0, 0)),
            pl.BlockSpec((1, N_E), lambda i: (0, 0)),
        ],
        out_specs=pl.BlockSpec((1, 1, B_BLK), lambda i: (i, 0, 0)),
        out_shape=jax.ShapeDtypeStruct((n_blk, 1, B_BLK), jnp.int32),
    )(z_flat, zz, e_t, ee)
    return idx3.reshape(N_TOK)


def _sc_gather(embedding_weight, idx):
    info = plsc.get_sparse_core_info()
    nw = info.num_cores * info.num_subcores
    b_per_w = N_TOK // nw
    chunk = 128
    n_chunks = b_per_w // chunk
    mesh = plsc.VectorSubcoreMesh(core_axis_name="c", subcore_axis_name="s")

    @functools.partial(
        pl.kernel,
        mesh=mesh,
        out_type=jax.ShapeDtypeStruct((N_TOK, E_DIM), jnp.float32),
        scratch_types=[
            pltpu.VMEM((chunk,), jnp.int32),
            pltpu.VMEM((chunk, E_DIM), jnp.float32),
            pltpu.SemaphoreType.DMA,
        ],
    )
    def gather_kernel(table_hbm, idx_hbm, out_hbm, idx_v, rows_v, sem):
        wid = lax.axis_index("s") * info.num_cores + lax.axis_index("c")
        base = wid * b_per_w

        def body(c, _):
            off = base + c * chunk
            pltpu.sync_copy(idx_hbm.at[pl.ds(off, chunk)], idx_v)
            pltpu.async_copy(table_hbm.at[idx_v], rows_v, sem).wait()
            pltpu.sync_copy(rows_v, out_hbm.at[pl.ds(off, chunk)])
            return _

        lax.fori_loop(0, n_chunks, body, 0)

    return gather_kernel(embedding_weight, idx)


def _loss_body(z_ref, zq_ref, loss_ref):
    w = zq_ref[...] - z_ref[...]
    w2 = w * w
    loss_ref[...] = w2 + BETA * w2


def _loss(z_flat, zq_flat):
    rows = 2048
    n_blk = N_TOK // rows
    return pl.pallas_call(
        _loss_body,
        grid=(n_blk,),
        in_specs=[
            pl.BlockSpec((rows, E_DIM), lambda i: (i, 0)),
            pl.BlockSpec((rows, E_DIM), lambda i: (i, 0)),
        ],
        out_specs=pl.BlockSpec((rows, E_DIM), lambda i: (i, 0)),
        out_shape=jax.ShapeDtypeStruct((N_TOK, E_DIM), jnp.float32),
    )(z_flat, zq_flat)


def kernel(z, embedding_weight):
    z_flat = z.reshape(-1, E_DIM)
    zz = jnp.sum(z_flat**2, axis=1, keepdims=True)
    ee = jnp.sum(embedding_weight**2, axis=1)[None, :]
    e_t = embedding_weight.T.astype(jnp.bfloat16)
    idx = _distance_argmin(z_flat, zz, e_t, ee)
    zq_flat = _sc_gather(embedding_weight, idx)
    loss = _loss(z_flat, zq_flat)
    return (zq_flat.reshape(z.shape), idx, loss.reshape(z.shape))


# speedup vs baseline: 1.1159x; 1.0249x over previous
import functools

import jax
import jax.numpy as jnp
from jax import lax
from jax.experimental import pallas as pl
from jax.experimental.pallas import tpu as pltpu
from jax.experimental.pallas import tpu_sc as plsc

N_E = 8192
E_DIM = 256
BETA = 0.25
N_TOK = 16384
B_BLK = 512


_WINDOWS = ((0, 2736), (2736, 5472), (5472, 8192))


def _bf16(x):
    return x.astype(jnp.bfloat16).astype(jnp.float32)


def _argmin_body(z_ref, zz_ref, et_ref, ee_ref, idx_ref):
    zb = z_ref[...].astype(jnp.bfloat16)
    mm = jnp.dot(zb, et_ref[...], preferred_element_type=jnp.float32)
    d = (zz_ref[...] + ee_ref[...]) - 2.0 * mm
    (l1, h1), (l2, h2), (l3, h3) = _WINDOWS
    m1 = jnp.min(d[:, l1:h1], axis=1, keepdims=True)
    m2 = jnp.min(d[:, l2:h2], axis=1, keepdims=True)
    m3 = jnp.min(d[:, l3:h3], axis=1, keepdims=True)
    b1 = _bf16(m1)
    t2 = m2 < b1
    b2 = _bf16(jnp.where(t2, m2, b1))
    t3 = m3 < b2
    niota = lax.broadcasted_iota(jnp.int32, d.shape, 1)
    mstar = jnp.where(niota < h1, m1, jnp.where(niota < h2, m2, m3))
    key = jnp.where(d == mstar, niota, N_E)
    i1 = jnp.min(key[:, l1:h1], axis=1, keepdims=True)
    i2 = jnp.min(key[:, l2:h2], axis=1, keepdims=True)
    i3 = jnp.min(key[:, l3:h3], axis=1, keepdims=True)
    idx = jnp.where(t3, i3, jnp.where(t2, i2, i1))
    idx_ref[0, 0, :] = idx[:, 0]


def _distance_argmin(z_flat, zz, e_t, ee):
    n_blk = N_TOK // B_BLK
    idx3 = pl.pallas_call(
        _argmin_body,
        grid=(n_blk,),
        in_specs=[
            pl.BlockSpec((B_BLK, E_DIM), lambda i: (i, 0)),
            pl.BlockSpec((B_BLK, 1), lambda i: (i, 0)),
            pl.BlockSpec((E_DIM, N_E), lambda i: (0, 0)),
            pl.BlockSpec((1, N_E), lambda i: (0, 0)),
        ],
        out_specs=pl.BlockSpec((1, 1, B_BLK), lambda i: (i, 0, 0)),
        out_shape=jax.ShapeDtypeStruct((n_blk, 1, B_BLK), jnp.int32),
    )(z_flat, zz, e_t, ee)
    return idx3.reshape(N_TOK)


def _sc_gather(embedding_weight, idx):
    info = plsc.get_sparse_core_info()
    nw = info.num_cores * info.num_subcores
    b_per_w = N_TOK // nw
    chunk = 128
    n_chunks = b_per_w // chunk
    mesh = plsc.VectorSubcoreMesh(core_axis_name="c", subcore_axis_name="s")

    @functools.partial(
        pl.kernel,
        mesh=mesh,
        out_type=jax.ShapeDtypeStruct((N_TOK, E_DIM), jnp.float32),
        scratch_types=[
            pltpu.VMEM((chunk,), jnp.int32),
            pltpu.VMEM((chunk, E_DIM), jnp.float32),
            pltpu.SemaphoreType.DMA,
        ],
    )
    def gather_kernel(table_hbm, idx_hbm, out_hbm, idx_v, rows_v, sem):
        wid = lax.axis_index("s") * info.num_cores + lax.axis_index("c")
        base = wid * b_per_w

        def body(c, _):
            off = base + c * chunk
            pltpu.sync_copy(idx_hbm.at[pl.ds(off, chunk)], idx_v)
            pltpu.async_copy(table_hbm.at[idx_v], rows_v, sem).wait()
            pltpu.sync_copy(rows_v, out_hbm.at[pl.ds(off, chunk)])
            return _

        lax.fori_loop(0, n_chunks, body, 0)

    return gather_kernel(embedding_weight, idx)


def _loss_body(z_ref, zq_ref, loss_ref):
    w = zq_ref[...] - z_ref[...]
    w2 = w * w
    loss_ref[...] = w2 + BETA * w2


def _loss(z_flat, zq_flat):
    rows = 2048
    n_blk = N_TOK // rows
    return pl.pallas_call(
        _loss_body,
        grid=(n_blk,),
        in_specs=[
            pl.BlockSpec((rows, E_DIM), lambda i: (i, 0)),
            pl.BlockSpec((rows, E_DIM), lambda i: (i, 0)),
        ],
        out_specs=pl.BlockSpec((rows, E_DIM), lambda i: (i, 0)),
        out_shape=jax.ShapeDtypeStruct((N_TOK, E_DIM), jnp.float32),
    )(z_flat, zq_flat)


def kernel(z, embedding_weight):
    z_flat = z.reshape(-1, E_DIM)
    zz = jnp.sum(z_flat**2, axis=1, keepdims=True)
    ee = jnp.sum(embedding_weight**2, axis=1)[None, :]
    e_t = embedding_weight.T.astype(jnp.bfloat16)
    idx = _distance_argmin(z_flat, zz, e_t, ee)
    zq_flat = _sc_gather(embedding_weight, idx)
    loss = _loss(z_flat, zq_flat)
    return (zq_flat.reshape(z.shape), idx, loss.reshape(z.shape))


# speedup vs baseline: 1.1676x; 1.0463x over previous
import functools

import jax
import jax.numpy as jnp
from jax import lax
from jax.experimental import pallas as pl
from jax.experimental.pallas import tpu as pltpu
from jax.experimental.pallas import tpu_sc as plsc

N_E = 8192
E_DIM = 256
BETA = 0.25
N_TOK = 16384
B_BLK = 1024


_WINDOWS = ((0, 2736), (2736, 5472), (5472, 8192))


def _bf16(x):
    return x.astype(jnp.bfloat16).astype(jnp.float32)


def _argmin_body(z_ref, zz_ref, et_ref, ee_ref, idx_ref):
    zb = z_ref[...].astype(jnp.bfloat16)
    mm = jnp.dot(zb, et_ref[...], preferred_element_type=jnp.float32)
    d = (zz_ref[...] + ee_ref[...]) - 2.0 * mm
    (l1, h1), (l2, h2), (l3, h3) = _WINDOWS
    m1 = jnp.min(d[:, l1:h1], axis=1, keepdims=True)
    m2 = jnp.min(d[:, l2:h2], axis=1, keepdims=True)
    m3 = jnp.min(d[:, l3:h3], axis=1, keepdims=True)
    b1 = _bf16(m1)
    t2 = m2 < b1
    b2 = _bf16(jnp.where(t2, m2, b1))
    t3 = m3 < b2
    niota = lax.broadcasted_iota(jnp.int32, d.shape, 1)
    mstar = jnp.where(niota < h1, m1, jnp.where(niota < h2, m2, m3))
    key = jnp.where(d == mstar, niota, N_E)
    i1 = jnp.min(key[:, l1:h1], axis=1, keepdims=True)
    i2 = jnp.min(key[:, l2:h2], axis=1, keepdims=True)
    i3 = jnp.min(key[:, l3:h3], axis=1, keepdims=True)
    idx = jnp.where(t3, i3, jnp.where(t2, i2, i1))
    idx_ref[0, 0, :] = idx[:, 0]


def _distance_argmin(z_flat, zz, e_t, ee):
    n_blk = N_TOK // B_BLK
    idx3 = pl.pallas_call(
        _argmin_body,
        grid=(n_blk,),
        in_specs=[
            pl.BlockSpec((B_BLK, E_DIM), lambda i: (i, 0)),
            pl.BlockSpec((B_BLK, 1), lambda i: (i, 0)),
            pl.BlockSpec((E_DIM, N_E), lambda i: (0, 0)),
            pl.BlockSpec((1, N_E), lambda i: (0, 0)),
        ],
        out_specs=pl.BlockSpec((1, 1, B_BLK), lambda i: (i, 0, 0)),
        out_shape=jax.ShapeDtypeStruct((n_blk, 1, B_BLK), jnp.int32),
    )(z_flat, zz, e_t, ee)
    return idx3.reshape(N_TOK)


def _sc_gather(embedding_weight, idx):
    info = plsc.get_sparse_core_info()
    nw = info.num_cores * info.num_subcores
    b_per_w = N_TOK // nw
    chunk = 128
    n_chunks = b_per_w // chunk
    mesh = plsc.VectorSubcoreMesh(core_axis_name="c", subcore_axis_name="s")

    @functools.partial(
        pl.kernel,
        mesh=mesh,
        out_type=jax.ShapeDtypeStruct((N_TOK, E_DIM), jnp.float32),
        scratch_types=[
            pltpu.VMEM((chunk,), jnp.int32),
            pltpu.VMEM((chunk, E_DIM), jnp.float32),
            pltpu.SemaphoreType.DMA,
        ],
    )
    def gather_kernel(table_hbm, idx_hbm, out_hbm, idx_v, rows_v, sem):
        wid = lax.axis_index("s") * info.num_cores + lax.axis_index("c")
        base = wid * b_per_w

        def body(c, _):
            off = base + c * chunk
            pltpu.sync_copy(idx_hbm.at[pl.ds(off, chunk)], idx_v)
            pltpu.async_copy(table_hbm.at[idx_v], rows_v, sem).wait()
            pltpu.sync_copy(rows_v, out_hbm.at[pl.ds(off, chunk)])
            return _

        lax.fori_loop(0, n_chunks, body, 0)

    return gather_kernel(embedding_weight, idx)


def _loss_body(z_ref, zq_ref, loss_ref):
    w = zq_ref[...] - z_ref[...]
    w2 = w * w
    loss_ref[...] = w2 + BETA * w2


def _loss(z_flat, zq_flat):
    rows = 2048
    n_blk = N_TOK // rows
    return pl.pallas_call(
        _loss_body,
        grid=(n_blk,),
        in_specs=[
            pl.BlockSpec((rows, E_DIM), lambda i: (i, 0)),
            pl.BlockSpec((rows, E_DIM), lambda i: (i, 0)),
        ],
        out_specs=pl.BlockSpec((rows, E_DIM), lambda i: (i, 0)),
        out_shape=jax.ShapeDtypeStruct((N_TOK, E_DIM), jnp.float32),
    )(z_flat, zq_flat)


def kernel(z, embedding_weight):
    z_flat = z.reshape(-1, E_DIM)
    zz = jnp.sum(z_flat**2, axis=1, keepdims=True)
    ee = jnp.sum(embedding_weight**2, axis=1)[None, :]
    e_t = embedding_weight.T.astype(jnp.bfloat16)
    idx = _distance_argmin(z_flat, zz, e_t, ee)
    zq_flat = _sc_gather(embedding_weight, idx)
    loss = _loss(z_flat, zq_flat)
    return (zq_flat.reshape(z.shape), idx, loss.reshape(z.shape))
